# baseline (speedup 1.0000x reference)
import functools

import jax
import jax.numpy as jnp
from jax import lax
from jax.experimental import pallas as pl
from jax.experimental.pallas import tpu as pltpu
from jax.experimental.pallas import tpu_sc as plsc

B, N, F, K = 8, 2048, 128, 20
P = N * F
CH = 8192
NCHUNK = P // CH
NK = N * K
RQ = 1024


def _topk_body(pts_ref, q_ref, idx_ref):
    pts = pts_ref[0]
    qc = q_ref[0]
    pxy = pts[:, 0:2]
    qxy = qc[0:2, :]
    inner = lax.dot_general(pxy, qxy, (((1,), (0,)), ((), ())),
                            preferred_element_type=jnp.float32)
    qxx = jnp.sum(qxy * qxy, axis=0, keepdims=True)
    pxx = jnp.sum(pxy * pxy, axis=1, keepdims=True)
    d = (qxx - 2.0 * inner) + pxx
    nch = N // 128
    ds = [d[s * 128:(s + 1) * 128, :] for s in range(nch)]
    m = jnp.concatenate(
        [jnp.min(ds[s], axis=0, keepdims=True) for s in range(nch)], axis=0)
    iota_c = lax.broadcasted_iota(jnp.int32, (nch, RQ), 0)
    iota_s = lax.broadcasted_iota(jnp.int32, (128, RQ), 0)
    vprev = jnp.full((1, RQ), -jnp.inf, jnp.float32)
    gprev = jnp.full((1, RQ), -1, jnp.int32)
    cols = []
    for j in range(K):
        mv = jnp.min(m, axis=0, keepdims=True)
        cb = jnp.min(jnp.where(m == mv, iota_c, nch), axis=0,
                     keepdims=True)
        sel = ds[0]
        for s in range(1, nch):
            sel = jnp.where(cb == s, ds[s], sel)
        iotag = iota_s + cb * 128
        hits = (sel == mv) & ((mv > vprev) | (iotag > gprev))
        col = jnp.min(jnp.where(hits, iotag, jnp.int32(1) << 30), axis=0,
                      keepdims=True)
        cols.append(col)
        if j < K - 1:
            live = (sel > mv) | ((sel == mv) & (iotag > col))
            nm = jnp.min(jnp.where(live, sel, jnp.inf), axis=0, keepdims=True)
            m = jnp.where(iota_c == cb, nm, m)
            vprev, gprev = mv, col
    idx_ref[0] = jnp.concatenate(cols, axis=0)


def _topk(xyz, xyz_t):
    nb = xyz.shape[0]
    return pl.pallas_call(
        _topk_body,
        grid=(nb, N // RQ),
        in_specs=[pl.BlockSpec((1, N, 3), lambda b, r: (b, 0, 0)),
                  pl.BlockSpec((1, 3, RQ), lambda b, r: (b, 0, r))],
        out_specs=pl.BlockSpec((1, K, RQ), lambda b, r: (b, 0, r)),
        out_shape=jax.ShapeDtypeStruct((nb, K, N), jnp.int32),
    )(xyz_t, xyz)


def _slot_geom(m, c):
    x = 3 * m + 2 * c
    a0 = x - 5 * ((x * 52429) >> 18)
    fb = (32 * a0 + c)
    fb = (fb * 52429) >> 18
    return a0, fb


def _sc_combine(feat, fidx, wtab):
    nb = feat.shape[0]
    mesh = plsc.VectorSubcoreMesh(core_axis_name="c", subcore_axis_name="s")

    @functools.partial(
        pl.kernel,
        mesh=mesh,
        out_type=jax.ShapeDtypeStruct((nb, P), jnp.float32),
        scratch_types=[
            pltpu.VMEM((NK,), jnp.int32),
            pltpu.VMEM((K * N,), jnp.float32),
            pltpu.VMEM(((K + 1) * 16,), jnp.float32),
            pltpu.VMEM((CH,), jnp.float32),
            pltpu.SemaphoreType.DMA,
        ],
        compiler_params=pltpu.CompilerParams(needs_layout_passes=False),
    )
    def run(feat_hbm, fidx_hbm, w_hbm, out_hbm, idx_v, rows_v, w_v, acc_v, sem):
        c = lax.axis_index("s") * 2 + lax.axis_index("c")
        for m in range(5):
            a0, _ = _slot_geom(m, c)
            for t in range(4):
                pltpu.sync_copy(w_hbm.at[pl.ds((a0 + 5 * t) * 16, 16)],
                                w_v.at[pl.ds((4 * m + t) * 16, 16)])
        pltpu.sync_copy(w_hbm.at[pl.ds(K * 16, 16)], w_v.at[pl.ds(K * 16, 16)])
        bias_v = w_v[pl.ds(K * 16, 16)]
        wvecs = [w_v[pl.ds(s * 16, 16)] for s in range(K)]

        def batch_body(b, carry):
            cps = [pltpu.async_copy(fidx_hbm.at[b], idx_v, sem)]
            for m in range(5):
                _, fb = _slot_geom(m, c)
                for t in range(4):
                    dst = rows_v.at[pl.ds((4 * m + t) * N, N)]
                    cps.append(
                        pltpu.async_copy(feat_hbm.at[b, fb + 32 * t], dst, sem))
            for cp in cps:
                cp.wait()

            def inner(i, _):
                u = i * 16
                acc = bias_v
                for m in range(5):
                    iv = idx_v[pl.ds(m * CH + u, 16)]
                    for t in range(4):
                        slot = 4 * m + t
                        g = plsc.load_gather(rows_v, [iv + slot * N])
                        acc = acc + g * wvecs[slot]
                acc_v[pl.ds(u, 16)] = acc
                return 0

            lax.fori_loop(0, CH // 16, inner, 0, unroll=4)
            pltpu.sync_copy(acc_v, out_hbm.at[b, pl.ds(c * CH, CH)])
            return 0

        lax.fori_loop(0, nb, batch_body, 0)

    return run(feat, fidx, wtab)


def _tr_body(x_ref, o_ref):
    o_ref[0] = x_ref[0].T


def _transpose(x):
    nb = x.shape[0]
    return pl.pallas_call(
        _tr_body,
        grid=(nb, N // 128),
        in_specs=[pl.BlockSpec((1, 128, F), lambda b, i: (b, i, 0))],
        out_specs=pl.BlockSpec((1, F, 128), lambda b, i: (b, 0, i)),
        out_shape=jax.ShapeDtypeStruct((nb, F, N), jnp.float32),
    )(x)


def kernel(xyz, feat, conv_w, conv_b):
    xyz = xyz.astype(jnp.float32)
    xyz_t = jnp.swapaxes(xyz, 1, 2)
    w = jnp.concatenate([conv_w.reshape(K), conv_b.reshape(1)])
    wtab = jnp.broadcast_to(w[:, None], (K + 1, 16)).reshape((K + 1) * 16)
    npiece = 4
    step = B // npiece
    pieces = []
    for lo in range(0, B, step):
        hi = lo + step
        nb = hi - lo
        idx = _topk(xyz[lo:hi], xyz_t[lo:hi])
        fidx = idx.transpose(0, 2, 1).reshape(nb, NK)
        out_nf = _sc_combine(feat[lo:hi], fidx, wtab)
        pieces.append(_transpose(out_nf.reshape(nb, N, F)))
    return jnp.concatenate(pieces, axis=0)


# speedup vs baseline: 1.0010x; 1.0010x over previous
import functools

import jax
import jax.numpy as jnp
from jax import lax
from jax.experimental import pallas as pl
from jax.experimental.pallas import tpu as pltpu
from jax.experimental.pallas import tpu_sc as plsc

B, N, F, K = 8, 2048, 128, 20
P = N * F
CH = 8192
NCHUNK = P // CH
NK = N * K
RQ = 1024


def _topk_body(pts_ref, q_ref, idx_ref):
    pts = pts_ref[0]
    qc = q_ref[0]
    pxy = pts[:, 0:2]
    qxy = qc[0:2, :]
    inner = lax.dot_general(pxy, qxy, (((1,), (0,)), ((), ())),
                            preferred_element_type=jnp.float32)
    qxx = jnp.sum(qxy * qxy, axis=0, keepdims=True)
    pxx = jnp.sum(pxy * pxy, axis=1, keepdims=True)
    d = (qxx - 2.0 * inner) + pxx
    nch = N // 128
    ds = [d[s * 128:(s + 1) * 128, :] for s in range(nch)]
    m = jnp.concatenate(
        [jnp.min(ds[s], axis=0, keepdims=True) for s in range(nch)], axis=0)
    iota_c = lax.broadcasted_iota(jnp.int32, (nch, RQ), 0)
    iota_s = lax.broadcasted_iota(jnp.int32, (128, RQ), 0)
    vprev = jnp.full((1, RQ), -jnp.inf, jnp.float32)
    gprev = jnp.full((1, RQ), -1, jnp.int32)
    cols = []
    for j in range(K):
        mv = jnp.min(m, axis=0, keepdims=True)
        cb = jnp.min(jnp.where(m == mv, iota_c, nch), axis=0,
                     keepdims=True)
        sel = ds[0]
        for s in range(1, nch):
            sel = jnp.where(cb == s, ds[s], sel)
        iotag = iota_s + cb * 128
        hits = (sel == mv) & ((mv > vprev) | (iotag > gprev))
        col = jnp.min(jnp.where(hits, iotag, jnp.int32(1) << 30), axis=0,
                      keepdims=True)
        cols.append(col)
        if j < K - 1:
            live = (sel > mv) | ((sel == mv) & (iotag > col))
            nm = jnp.min(jnp.where(live, sel, jnp.inf), axis=0, keepdims=True)
            m = jnp.where(iota_c == cb, nm, m)
            vprev, gprev = mv, col
    idx_ref[0] = jnp.concatenate(cols, axis=0)


def _topk(xyz, xyz_t):
    nb = xyz.shape[0]
    return pl.pallas_call(
        _topk_body,
        grid=(nb, N // RQ),
        in_specs=[pl.BlockSpec((1, N, 3), lambda b, r: (b, 0, 0)),
                  pl.BlockSpec((1, 3, RQ), lambda b, r: (b, 0, r))],
        out_specs=pl.BlockSpec((1, K, RQ), lambda b, r: (b, 0, r)),
        out_shape=jax.ShapeDtypeStruct((nb, K, N), jnp.int32),
    )(xyz_t, xyz)


def _slot_geom(m, c):
    x = 3 * m + 2 * c
    a0 = x - 5 * ((x * 52429) >> 18)
    fb = (32 * a0 + c)
    fb = (fb * 52429) >> 18
    return a0, fb


def _sc_combine(feat, fidx, wtab):
    nb = feat.shape[0]
    mesh = plsc.VectorSubcoreMesh(core_axis_name="c", subcore_axis_name="s")

    @functools.partial(
        pl.kernel,
        mesh=mesh,
        out_type=jax.ShapeDtypeStruct((nb, P), jnp.float32),
        scratch_types=[
            pltpu.VMEM((NK,), jnp.int32),
            pltpu.VMEM((K * N,), jnp.float32),
            pltpu.VMEM(((K + 1) * 16,), jnp.float32),
            pltpu.VMEM((CH,), jnp.float32),
            pltpu.SemaphoreType.DMA,
        ],
        compiler_params=pltpu.CompilerParams(needs_layout_passes=False),
    )
    def run(feat_hbm, fidx_hbm, w_hbm, out_hbm, idx_v, rows_v, w_v, acc_v, sem):
        c = lax.axis_index("s") * 2 + lax.axis_index("c")
        for m in range(5):
            a0, _ = _slot_geom(m, c)
            for t in range(4):
                pltpu.sync_copy(w_hbm.at[pl.ds((a0 + 5 * t) * 16, 16)],
                                w_v.at[pl.ds((4 * m + t) * 16, 16)])
        pltpu.sync_copy(w_hbm.at[pl.ds(K * 16, 16)], w_v.at[pl.ds(K * 16, 16)])
        bias_v = w_v[pl.ds(K * 16, 16)]
        wvecs = [w_v[pl.ds(s * 16, 16)] for s in range(K)]

        def batch_body(b, carry):
            cps = [pltpu.async_copy(fidx_hbm.at[b], idx_v, sem)]
            for m in range(5):
                _, fb = _slot_geom(m, c)
                for t in range(4):
                    dst = rows_v.at[pl.ds((4 * m + t) * N, N)]
                    cps.append(
                        pltpu.async_copy(feat_hbm.at[b, fb + 32 * t], dst, sem))
            for cp in cps:
                cp.wait()

            def inner(i, _):
                u = i * 16
                acc = bias_v
                for m in range(5):
                    iv = idx_v[pl.ds(m * CH + u, 16)]
                    for t in range(4):
                        slot = 4 * m + t
                        g = plsc.load_gather(rows_v, [iv + slot * N])
                        acc = acc + g * wvecs[slot]
                acc_v[pl.ds(u, 16)] = acc
                return 0

            lax.fori_loop(0, CH // 16, inner, 0, unroll=2)
            pltpu.sync_copy(acc_v, out_hbm.at[b, pl.ds(c * CH, CH)])
            return 0

        lax.fori_loop(0, nb, batch_body, 0)

    return run(feat, fidx, wtab)


def _tr_body(x_ref, o_ref):
    o_ref[0] = x_ref[0].T


def _transpose(x):
    nb = x.shape[0]
    return pl.pallas_call(
        _tr_body,
        grid=(nb, N // 128),
        in_specs=[pl.BlockSpec((1, 128, F), lambda b, i: (b, i, 0))],
        out_specs=pl.BlockSpec((1, F, 128), lambda b, i: (b, 0, i)),
        out_shape=jax.ShapeDtypeStruct((nb, F, N), jnp.float32),
    )(x)


def kernel(xyz, feat, conv_w, conv_b):
    xyz = xyz.astype(jnp.float32)
    xyz_t = jnp.swapaxes(xyz, 1, 2)
    w = jnp.concatenate([conv_w.reshape(K), conv_b.reshape(1)])
    wtab = jnp.broadcast_to(w[:, None], (K + 1, 16)).reshape((K + 1) * 16)
    npiece = 4
    step = B // npiece
    pieces = []
    for lo in range(0, B, step):
        hi = lo + step
        nb = hi - lo
        idx = _topk(xyz[lo:hi], xyz_t[lo:hi])
        fidx = idx.transpose(0, 2, 1).reshape(nb, NK)
        out_nf = _sc_combine(feat[lo:hi], fidx, wtab)
        pieces.append(_transpose(out_nf.reshape(nb, N, F)))
    return jnp.concatenate(pieces, axis=0)
